# baseline (device time: 14103 ns/iter reference)
import jax
import jax.numpy as jnp
from jax import lax
from jax.experimental import pallas as pl
from jax.experimental.pallas import tpu as pltpu

N_DEV = 32
B = 256
H = 8


def kernel(x):
    m, n = x.shape
    nb = m // B

    def body(x_hbm, out_hbm, xb, ob, halo_prev, halo_next,
             in_sems, out_sems, send_sems, recv_sems):
        my_i = lax.axis_index("i")
        has_left = my_i > 0
        has_right = my_i < N_DEV - 1

        barrier_sem = pltpu.get_barrier_semaphore()

        @pl.when(has_left)
        def _():
            pl.semaphore_signal(
                barrier_sem, inc=1,
                device_id=(my_i - 1,), device_id_type=pl.DeviceIdType.MESH,
            )

        @pl.when(has_right)
        def _():
            pl.semaphore_signal(
                barrier_sem, inc=1,
                device_id=(my_i + 1,), device_id_type=pl.DeviceIdType.MESH,
            )

        @pl.when(has_left)
        def _():
            pl.semaphore_wait(barrier_sem, 1)

        @pl.when(has_right)
        def _():
            pl.semaphore_wait(barrier_sem, 1)

        @pl.when(has_right)
        def _():
            pltpu.make_async_remote_copy(
                src_ref=x_hbm.at[pl.ds(m - H, H), :],
                dst_ref=halo_prev,
                send_sem=send_sems.at[0],
                recv_sem=recv_sems.at[0],
                device_id=(my_i + 1,),
                device_id_type=pl.DeviceIdType.MESH,
            ).start()

        @pl.when(has_left)
        def _():
            pltpu.make_async_remote_copy(
                src_ref=x_hbm.at[pl.ds(0, H), :],
                dst_ref=halo_next,
                send_sem=send_sems.at[1],
                recv_sem=recv_sems.at[1],
                device_id=(my_i - 1,),
                device_id_type=pl.DeviceIdType.MESH,
            ).start()

        def start_in(b, slot):
            if b == 0:
                d = pltpu.make_async_copy(
                    x_hbm.at[pl.ds(0, B + H), :],
                    xb.at[slot, pl.ds(H, B + H), :],
                    in_sems.at[slot],
                )
            elif b == nb - 1:
                d = pltpu.make_async_copy(
                    x_hbm.at[pl.ds(m - B - H, B + H), :],
                    xb.at[slot, pl.ds(0, B + H), :],
                    in_sems.at[slot],
                )
            else:
                d = pltpu.make_async_copy(
                    x_hbm.at[pl.ds(b * B - H, B + 2 * H), :],
                    xb.at[slot],
                    in_sems.at[slot],
                )
            d.start()
            return d

        order = list(range(1, nb)) + [0]
        in_flight = {0: start_in(order[0], 0)}
        out_flight = {}

        for k, b in enumerate(order):
            slot = k % 2
            if k + 1 < nb:
                in_flight[(k + 1) % 2] = start_in(order[k + 1], (k + 1) % 2)
            in_flight[slot].wait()

            if b == nb - 1:
                @pl.when(has_right)
                def _():
                    pltpu.make_async_remote_copy(
                        src_ref=x_hbm.at[pl.ds(0, H), :],
                        dst_ref=halo_next,
                        send_sem=send_sems.at[1],
                        recv_sem=recv_sems.at[1],
                        device_id=(my_i + 1,),
                        device_id_type=pl.DeviceIdType.MESH,
                    ).wait_recv()
            if b == 0:
                @pl.when(has_left)
                def _():
                    pltpu.make_async_remote_copy(
                        src_ref=x_hbm.at[pl.ds(m - H, H), :],
                        dst_ref=halo_prev,
                        send_sem=send_sems.at[0],
                        recv_sem=recv_sems.at[0],
                        device_id=(my_i - 1,),
                        device_id_type=pl.DeviceIdType.MESH,
                    ).wait_recv()

            if slot in out_flight:
                out_flight[slot].wait()

            xv = xb[slot, :, :]
            res = (
                0.25 * xv[H - 1 : B + H - 1]
                + 0.5 * xv[H : B + H]
                + 0.25 * xv[H + 1 : B + H + 1]
            )
            ob[slot, :, :] = res
            if b == 0:
                c = xv[H : H + 1, :]
                r0 = 0.25 * halo_prev[H - 1 : H, :] + 0.5 * c \
                    + 0.25 * xv[H + 1 : H + 2, :]
                ob[slot, 0:1, :] = jnp.where(my_i == 0, c, r0)
            if b == nb - 1:
                c = xv[B + H - 1 : B + H, :]
                rl = 0.25 * xv[B + H - 2 : B + H - 1, :] + 0.5 * c \
                    + 0.25 * halo_next[0:1, :]
                ob[slot, B - 1 : B, :] = jnp.where(my_i == N_DEV - 1, c, rl)

            d = pltpu.make_async_copy(
                ob.at[slot],
                out_hbm.at[pl.ds(b * B, B), :],
                out_sems.at[slot],
            )
            d.start()
            out_flight[slot] = d

        for slot in sorted(out_flight):
            out_flight[slot].wait()

        @pl.when(has_right)
        def _():
            pltpu.make_async_remote_copy(
                src_ref=x_hbm.at[pl.ds(m - H, H), :],
                dst_ref=halo_prev,
                send_sem=send_sems.at[0],
                recv_sem=recv_sems.at[0],
                device_id=(my_i + 1,),
                device_id_type=pl.DeviceIdType.MESH,
            ).wait_send()

        @pl.when(has_left)
        def _():
            pltpu.make_async_remote_copy(
                src_ref=x_hbm.at[pl.ds(0, H), :],
                dst_ref=halo_next,
                send_sem=send_sems.at[1],
                recv_sem=recv_sems.at[1],
                device_id=(my_i - 1,),
                device_id_type=pl.DeviceIdType.MESH,
            ).wait_send()

    return pl.pallas_call(
        body,
        out_shape=jax.ShapeDtypeStruct((m, n), x.dtype),
        in_specs=[pl.BlockSpec(memory_space=pltpu.MemorySpace.HBM)],
        out_specs=pl.BlockSpec(memory_space=pltpu.MemorySpace.HBM),
        scratch_shapes=[
            pltpu.VMEM((2, B + 2 * H, n), jnp.float32),
            pltpu.VMEM((2, B, n), jnp.float32),
            pltpu.VMEM((H, n), jnp.float32),
            pltpu.VMEM((H, n), jnp.float32),
            pltpu.SemaphoreType.DMA((2,)),
            pltpu.SemaphoreType.DMA((2,)),
            pltpu.SemaphoreType.DMA((2,)),
            pltpu.SemaphoreType.DMA((2,)),
        ],
        compiler_params=pltpu.CompilerParams(collective_id=0),
    )(x)


# device time: 13864 ns/iter; 1.0172x vs baseline; 1.0172x over previous
import jax
import jax.numpy as jnp
from jax import lax
from jax.experimental import pallas as pl
from jax.experimental.pallas import tpu as pltpu

N_DEV = 32
B = 512
H = 8


def kernel(x):
    m, n = x.shape
    nb = m // B

    def body(x_hbm, out_hbm, xb, ob, halo_prev, halo_next,
             in_sems, out_sems, send_sems, recv_sems):
        my_i = lax.axis_index("i")
        has_left = my_i > 0
        has_right = my_i < N_DEV - 1

        barrier_sem = pltpu.get_barrier_semaphore()

        @pl.when(has_left)
        def _():
            pl.semaphore_signal(
                barrier_sem, inc=1,
                device_id=(my_i - 1,), device_id_type=pl.DeviceIdType.MESH,
            )

        @pl.when(has_right)
        def _():
            pl.semaphore_signal(
                barrier_sem, inc=1,
                device_id=(my_i + 1,), device_id_type=pl.DeviceIdType.MESH,
            )

        @pl.when(has_left)
        def _():
            pl.semaphore_wait(barrier_sem, 1)

        @pl.when(has_right)
        def _():
            pl.semaphore_wait(barrier_sem, 1)

        @pl.when(has_right)
        def _():
            pltpu.make_async_remote_copy(
                src_ref=x_hbm.at[pl.ds(m - H, H), :],
                dst_ref=halo_prev,
                send_sem=send_sems.at[0],
                recv_sem=recv_sems.at[0],
                device_id=(my_i + 1,),
                device_id_type=pl.DeviceIdType.MESH,
            ).start()

        @pl.when(has_left)
        def _():
            pltpu.make_async_remote_copy(
                src_ref=x_hbm.at[pl.ds(0, H), :],
                dst_ref=halo_next,
                send_sem=send_sems.at[1],
                recv_sem=recv_sems.at[1],
                device_id=(my_i - 1,),
                device_id_type=pl.DeviceIdType.MESH,
            ).start()

        def start_in(b, slot):
            if b == 0:
                d = pltpu.make_async_copy(
                    x_hbm.at[pl.ds(0, B + H), :],
                    xb.at[slot, pl.ds(H, B + H), :],
                    in_sems.at[slot],
                )
            elif b == nb - 1:
                d = pltpu.make_async_copy(
                    x_hbm.at[pl.ds(m - B - H, B + H), :],
                    xb.at[slot, pl.ds(0, B + H), :],
                    in_sems.at[slot],
                )
            else:
                d = pltpu.make_async_copy(
                    x_hbm.at[pl.ds(b * B - H, B + 2 * H), :],
                    xb.at[slot],
                    in_sems.at[slot],
                )
            d.start()
            return d

        order = list(range(1, nb)) + [0]
        in_flight = {0: start_in(order[0], 0)}
        out_flight = {}

        for k, b in enumerate(order):
            slot = k % 2
            if k + 1 < nb:
                in_flight[(k + 1) % 2] = start_in(order[k + 1], (k + 1) % 2)
            in_flight[slot].wait()

            if b == nb - 1:
                @pl.when(has_right)
                def _():
                    pltpu.make_async_remote_copy(
                        src_ref=x_hbm.at[pl.ds(0, H), :],
                        dst_ref=halo_next,
                        send_sem=send_sems.at[1],
                        recv_sem=recv_sems.at[1],
                        device_id=(my_i + 1,),
                        device_id_type=pl.DeviceIdType.MESH,
                    ).wait_recv()
            if b == 0:
                @pl.when(has_left)
                def _():
                    pltpu.make_async_remote_copy(
                        src_ref=x_hbm.at[pl.ds(m - H, H), :],
                        dst_ref=halo_prev,
                        send_sem=send_sems.at[0],
                        recv_sem=recv_sems.at[0],
                        device_id=(my_i - 1,),
                        device_id_type=pl.DeviceIdType.MESH,
                    ).wait_recv()

            if slot in out_flight:
                out_flight[slot].wait()

            xv = xb[slot, :, :]
            res = (
                0.25 * xv[H - 1 : B + H - 1]
                + 0.5 * xv[H : B + H]
                + 0.25 * xv[H + 1 : B + H + 1]
            )
            ob[slot, :, :] = res
            if b == 0:
                c = xv[H : H + 1, :]
                r0 = 0.25 * halo_prev[H - 1 : H, :] + 0.5 * c \
                    + 0.25 * xv[H + 1 : H + 2, :]
                ob[slot, 0:1, :] = jnp.where(my_i == 0, c, r0)
            if b == nb - 1:
                c = xv[B + H - 1 : B + H, :]
                rl = 0.25 * xv[B + H - 2 : B + H - 1, :] + 0.5 * c \
                    + 0.25 * halo_next[0:1, :]
                ob[slot, B - 1 : B, :] = jnp.where(my_i == N_DEV - 1, c, rl)

            d = pltpu.make_async_copy(
                ob.at[slot],
                out_hbm.at[pl.ds(b * B, B), :],
                out_sems.at[slot],
            )
            d.start()
            out_flight[slot] = d

        for slot in sorted(out_flight):
            out_flight[slot].wait()

        @pl.when(has_right)
        def _():
            pltpu.make_async_remote_copy(
                src_ref=x_hbm.at[pl.ds(m - H, H), :],
                dst_ref=halo_prev,
                send_sem=send_sems.at[0],
                recv_sem=recv_sems.at[0],
                device_id=(my_i + 1,),
                device_id_type=pl.DeviceIdType.MESH,
            ).wait_send()

        @pl.when(has_left)
        def _():
            pltpu.make_async_remote_copy(
                src_ref=x_hbm.at[pl.ds(0, H), :],
                dst_ref=halo_next,
                send_sem=send_sems.at[1],
                recv_sem=recv_sems.at[1],
                device_id=(my_i - 1,),
                device_id_type=pl.DeviceIdType.MESH,
            ).wait_send()

    return pl.pallas_call(
        body,
        out_shape=jax.ShapeDtypeStruct((m, n), x.dtype),
        in_specs=[pl.BlockSpec(memory_space=pltpu.MemorySpace.HBM)],
        out_specs=pl.BlockSpec(memory_space=pltpu.MemorySpace.HBM),
        scratch_shapes=[
            pltpu.VMEM((2, B + 2 * H, n), jnp.float32),
            pltpu.VMEM((2, B, n), jnp.float32),
            pltpu.VMEM((H, n), jnp.float32),
            pltpu.VMEM((H, n), jnp.float32),
            pltpu.SemaphoreType.DMA((2,)),
            pltpu.SemaphoreType.DMA((2,)),
            pltpu.SemaphoreType.DMA((2,)),
            pltpu.SemaphoreType.DMA((2,)),
        ],
        compiler_params=pltpu.CompilerParams(collective_id=0),
    )(x)
